# baseline (speedup 1.0000x reference)
import functools

import jax
import jax.numpy as jnp
from jax import lax
from jax.experimental import pallas as pl
from jax.experimental.pallas import tpu as pltpu

N_CODES = 1024
DIM = 256
N_STAGES = 4
BETA_C = 0.25
BL = 1024
NH = 2
H = BL // NH
ROWS = 8192


def _vq_body(nsteps, z_ref, w_ref, zq_ref, enc_ref, idx_ref, loss_ref,
             ppl_ref, cnt_scr):
    i = pl.program_id(0)

    @pl.when(i == 0)
    def _init():
        loss_ref[...] = jnp.zeros_like(loss_ref)
        cnt_scr[...] = jnp.zeros_like(cnt_scr)

    w = w_ref[...]
    w2x = w + w
    wsq = lax.dot_general(jnp.ones((1, DIM), jnp.float32), w * w,
                          (((1,), (1,)), ((), ())),
                          preferred_element_type=jnp.float32,
                          precision=lax.Precision.HIGHEST)
    w1 = w.astype(jnp.bfloat16)
    w2 = (w - w1.astype(jnp.float32)).astype(jnp.bfloat16)
    w3 = (w - w1.astype(jnp.float32) - w2.astype(jnp.float32)).astype(jnp.bfloat16)

    iota = lax.broadcasted_iota(jnp.int32, (H, N_CODES), 1).astype(jnp.float32)
    z_rows = jnp.transpose(z_ref[0], (1, 0))
    residual = [z_rows[h * H:(h + 1) * H, :] for h in range(NH)]
    qsum = [jnp.zeros((H, DIM), jnp.float32) for _ in range(NH)]
    rsq = [jnp.sum(residual[h] * residual[h], axis=1, keepdims=True)
           for h in range(NH)]
    lsum = jnp.zeros((1, 1), jnp.float32)
    cnt = jnp.zeros((1, N_CODES), jnp.float32)
    idx_cols = [[] for _ in range(NH)]
    for q in range(N_STAGES):
        for h in range(NH):
            s2 = lax.dot_general(residual[h], w2x, (((1,), (1,)), ((), ())),
                                 preferred_element_type=jnp.float32)
            dist = (rsq[h] + wsq) - s2
            dmin = jnp.min(dist, axis=1, keepdims=True)
            idxm = jnp.min(jnp.where(dist <= dmin, iota, float(N_CODES)),
                           axis=1, keepdims=True)
            oh = (iota == idxm).astype(jnp.float32)
            enc_ref[q, pl.ds(h * H, H), :] = oh
            cnt = cnt + jnp.sum(oh, axis=0, keepdims=True)
            ohb = oh.astype(jnp.bfloat16)
            zqd = (lax.dot_general(ohb, w1, (((1,), (0,)), ((), ())),
                                   preferred_element_type=jnp.float32)
                   + lax.dot_general(ohb, w2, (((1,), (0,)), ((), ())),
                                     preferred_element_type=jnp.float32)
                   + lax.dot_general(ohb, w3, (((1,), (0,)), ((), ())),
                                     preferred_element_type=jnp.float32))
            qsum[h] = qsum[h] + zqd
            residual[h] = residual[h] - zqd
            rsq[h] = jnp.sum(residual[h] * residual[h], axis=1, keepdims=True)
            lsum = lsum + jnp.sum(rsq[h], axis=0, keepdims=True)
            idx_cols[h].append(idxm.astype(jnp.int32))

    zq_ref[0] = jnp.transpose(jnp.concatenate(qsum, axis=0), (1, 0))
    for h in range(NH):
        idx_ref[pl.ds(h * H, H), :] = jnp.concatenate(idx_cols[h], axis=1)
    loss_ref[...] += lsum
    cnt_scr[...] += cnt

    @pl.when(i == nsteps - 1)
    def _fini():
        loss_ref[...] = loss_ref[...] * (BETA_C / (nsteps * BL * DIM))
        avg = cnt_scr[...] * (1.0 / (nsteps * BL * N_STAGES))
        ent = jnp.sum(avg * jnp.log(avg + 1e-10), axis=1, keepdims=True)
        ppl_ref[...] = jnp.exp(-ent)


@jax.jit
def kernel(z, weight):
    b, c, h, w = z.shape
    nsteps = ROWS // BL
    z_cm = z.reshape(b, c, h * w)

    zq_cm, enc, idx, loss, ppl = pl.pallas_call(
        functools.partial(_vq_body, nsteps),
        grid=(nsteps,),
        in_specs=[
            pl.BlockSpec((1, DIM, BL), lambda i: (i, 0, 0)),
            pl.BlockSpec((N_CODES, DIM), lambda i: (0, 0)),
        ],
        out_specs=[
            pl.BlockSpec((1, DIM, BL), lambda i: (i, 0, 0)),
            pl.BlockSpec((N_STAGES, BL, N_CODES), lambda i: (0, i, 0)),
            pl.BlockSpec((BL, N_STAGES), lambda i: (i, 0)),
            pl.BlockSpec((1, 1), lambda i: (0, 0)),
            pl.BlockSpec((1, 1), lambda i: (0, 0)),
        ],
        out_shape=[
            jax.ShapeDtypeStruct((b, DIM, h * w), jnp.float32),
            jax.ShapeDtypeStruct((N_STAGES, ROWS, N_CODES), jnp.float32),
            jax.ShapeDtypeStruct((ROWS, N_STAGES), jnp.int32),
            jax.ShapeDtypeStruct((1, 1), jnp.float32),
            jax.ShapeDtypeStruct((1, 1), jnp.float32),
        ],
        scratch_shapes=[pltpu.VMEM((1, N_CODES), jnp.float32)],
    )(z_cm, weight)

    z_q = zq_cm.reshape(b, c, h, w)
    encodings_cat = enc.reshape(N_STAGES * ROWS, N_CODES)
    indices_stack = jnp.transpose(idx.reshape(b, h, w, N_STAGES), (0, 3, 1, 2))
    return (z_q, loss[0, 0], ppl[0, 0], encodings_cat, indices_stack)


# speedup vs baseline: 1.0074x; 1.0074x over previous
import functools

import jax
import jax.numpy as jnp
from jax import lax
from jax.experimental import pallas as pl
from jax.experimental.pallas import tpu as pltpu

N_CODES = 1024
DIM = 256
N_STAGES = 4
BETA_C = 0.25
BL = 512
NH = 2
H = BL // NH
ROWS = 8192


def _vq_body(nsteps, z_ref, w_ref, zq_ref, enc_ref, idx_ref, loss_ref,
             ppl_ref, cnt_scr):
    i = pl.program_id(0)

    @pl.when(i == 0)
    def _init():
        loss_ref[...] = jnp.zeros_like(loss_ref)
        cnt_scr[...] = jnp.zeros_like(cnt_scr)

    w = w_ref[...]
    w2x = w + w
    wsq = lax.dot_general(jnp.ones((1, DIM), jnp.float32), w * w,
                          (((1,), (1,)), ((), ())),
                          preferred_element_type=jnp.float32,
                          precision=lax.Precision.HIGHEST)
    w1 = w.astype(jnp.bfloat16)
    w2 = (w - w1.astype(jnp.float32)).astype(jnp.bfloat16)
    w3 = (w - w1.astype(jnp.float32) - w2.astype(jnp.float32)).astype(jnp.bfloat16)

    iota = lax.broadcasted_iota(jnp.int32, (H, N_CODES), 1).astype(jnp.float32)
    residual = [z_ref[pl.ds(h * H, H), :] for h in range(NH)]
    qsum = [jnp.zeros((H, DIM), jnp.float32) for _ in range(NH)]
    rsq = [jnp.sum(residual[h] * residual[h], axis=1, keepdims=True)
           for h in range(NH)]
    lsum = jnp.zeros((1, 1), jnp.float32)
    cnt = jnp.zeros((1, N_CODES), jnp.float32)
    idx_cols = [[] for _ in range(NH)]
    for q in range(N_STAGES):
        for h in range(NH):
            s2 = lax.dot_general(residual[h], w2x, (((1,), (1,)), ((), ())),
                                 preferred_element_type=jnp.float32)
            dist = (rsq[h] + wsq) - s2
            dmin = jnp.min(dist, axis=1, keepdims=True)
            idxm = jnp.min(jnp.where(dist <= dmin, iota, float(N_CODES)),
                           axis=1, keepdims=True)
            oh = (iota == idxm).astype(jnp.float32)
            enc_ref[q, pl.ds(h * H, H), :] = oh
            cnt = cnt + jnp.sum(oh, axis=0, keepdims=True)
            ohb = oh.astype(jnp.bfloat16)
            zqd = (lax.dot_general(ohb, w1, (((1,), (0,)), ((), ())),
                                   preferred_element_type=jnp.float32)
                   + lax.dot_general(ohb, w2, (((1,), (0,)), ((), ())),
                                     preferred_element_type=jnp.float32)
                   + lax.dot_general(ohb, w3, (((1,), (0,)), ((), ())),
                                     preferred_element_type=jnp.float32))
            qsum[h] = qsum[h] + zqd
            residual[h] = residual[h] - zqd
            rsq[h] = jnp.sum(residual[h] * residual[h], axis=1, keepdims=True)
            lsum = lsum + jnp.sum(rsq[h], axis=0, keepdims=True)
            idx_cols[h].append(idxm.astype(jnp.int32))

    for h in range(NH):
        zq_ref[pl.ds(h * H, H), :] = qsum[h]
        idx_ref[pl.ds(h * H, H), :] = jnp.concatenate(idx_cols[h], axis=1)
    loss_ref[...] += lsum
    cnt_scr[...] += cnt

    @pl.when(i == nsteps - 1)
    def _fini():
        loss_ref[...] = loss_ref[...] * (BETA_C / (nsteps * BL * DIM))
        avg = cnt_scr[...] * (1.0 / (nsteps * BL * N_STAGES))
        ent = jnp.sum(avg * jnp.log(avg + 1e-10), axis=1, keepdims=True)
        ppl_ref[...] = jnp.exp(-ent)


@jax.jit
def kernel(z, weight):
    b, c, h, w = z.shape
    nsteps = ROWS // BL
    z_flat = jnp.transpose(z, (0, 2, 3, 1)).reshape(ROWS, DIM)

    zq_flat, enc, idx, loss, ppl = pl.pallas_call(
        functools.partial(_vq_body, nsteps),
        grid=(nsteps,),
        in_specs=[
            pl.BlockSpec((BL, DIM), lambda i: (i, 0)),
            pl.BlockSpec((N_CODES, DIM), lambda i: (0, 0)),
        ],
        out_specs=[
            pl.BlockSpec((BL, DIM), lambda i: (i, 0)),
            pl.BlockSpec((N_STAGES, BL, N_CODES), lambda i: (0, i, 0)),
            pl.BlockSpec((BL, N_STAGES), lambda i: (i, 0)),
            pl.BlockSpec((1, 1), lambda i: (0, 0)),
            pl.BlockSpec((1, 1), lambda i: (0, 0)),
        ],
        out_shape=[
            jax.ShapeDtypeStruct((ROWS, DIM), jnp.float32),
            jax.ShapeDtypeStruct((N_STAGES, ROWS, N_CODES), jnp.float32),
            jax.ShapeDtypeStruct((ROWS, N_STAGES), jnp.int32),
            jax.ShapeDtypeStruct((1, 1), jnp.float32),
            jax.ShapeDtypeStruct((1, 1), jnp.float32),
        ],
        scratch_shapes=[pltpu.VMEM((1, N_CODES), jnp.float32)],
    )(z_flat, weight)

    z_q = jnp.transpose(zq_flat.reshape(b, h, w, DIM), (0, 3, 1, 2))
    encodings_cat = enc.reshape(N_STAGES * ROWS, N_CODES)
    indices_stack = jnp.transpose(idx.reshape(b, h, w, N_STAGES), (0, 3, 1, 2))
    return (z_q, loss[0, 0], ppl[0, 0], encodings_cat, indices_stack)


# speedup vs baseline: 1.1644x; 1.1559x over previous
import functools

import jax
import jax.numpy as jnp
from jax import lax
from jax.experimental import pallas as pl
from jax.experimental.pallas import tpu as pltpu

N_CODES = 1024
DIM = 256
N_STAGES = 4
BETA_C = 0.25
BL = 1024
NH = 2
H = BL // NH
ROWS = 8192


def _vq_body(nsteps, z_ref, w_ref, zq_ref, enc_ref, idx_ref, loss_ref,
             ppl_ref, cnt_scr):
    i = pl.program_id(0)

    @pl.when(i == 0)
    def _init():
        loss_ref[...] = jnp.zeros_like(loss_ref)
        cnt_scr[...] = jnp.zeros_like(cnt_scr)

    w = w_ref[...]
    w2x = w + w
    wsq = lax.dot_general(jnp.ones((1, DIM), jnp.float32), w * w,
                          (((1,), (1,)), ((), ())),
                          preferred_element_type=jnp.float32,
                          precision=lax.Precision.HIGHEST)
    w1 = w.astype(jnp.bfloat16)
    w2 = (w - w1.astype(jnp.float32)).astype(jnp.bfloat16)
    w3 = (w - w1.astype(jnp.float32) - w2.astype(jnp.float32)).astype(jnp.bfloat16)

    iota = lax.broadcasted_iota(jnp.int32, (H, N_CODES), 1).astype(jnp.float32)
    residual = [z_ref[pl.ds(h * H, H), :] for h in range(NH)]
    qsum = [jnp.zeros((H, DIM), jnp.float32) for _ in range(NH)]
    rsq = [jnp.sum(residual[h] * residual[h], axis=1, keepdims=True)
           for h in range(NH)]
    lsum = jnp.zeros((1, 1), jnp.float32)
    cnt = jnp.zeros((1, N_CODES), jnp.float32)
    idx_cols = [[] for _ in range(NH)]
    for q in range(N_STAGES):
        for h in range(NH):
            s2 = lax.dot_general(residual[h], w2x, (((1,), (1,)), ((), ())),
                                 preferred_element_type=jnp.float32)
            dist = (rsq[h] + wsq) - s2
            dmin = jnp.min(dist, axis=1, keepdims=True)
            idxm = jnp.min(jnp.where(dist <= dmin, iota, float(N_CODES)),
                           axis=1, keepdims=True)
            oh = (iota == idxm).astype(jnp.float32)
            enc_ref[q, pl.ds(h * H, H), :] = oh
            cnt = cnt + jnp.sum(oh, axis=0, keepdims=True)
            ohb = oh.astype(jnp.bfloat16)
            zqd = (lax.dot_general(ohb, w1, (((1,), (0,)), ((), ())),
                                   preferred_element_type=jnp.float32)
                   + lax.dot_general(ohb, w2, (((1,), (0,)), ((), ())),
                                     preferred_element_type=jnp.float32)
                   + lax.dot_general(ohb, w3, (((1,), (0,)), ((), ())),
                                     preferred_element_type=jnp.float32))
            qsum[h] = qsum[h] + zqd
            residual[h] = residual[h] - zqd
            rsq[h] = jnp.sum(residual[h] * residual[h], axis=1, keepdims=True)
            lsum = lsum + jnp.sum(rsq[h], axis=0, keepdims=True)
            idx_cols[h].append(idxm.astype(jnp.int32))

    for h in range(NH):
        zq_ref[pl.ds(h * H, H), :] = qsum[h]
        idx_ref[pl.ds(h * H, H), :] = jnp.concatenate(idx_cols[h], axis=1)
    loss_ref[...] += lsum
    cnt_scr[...] += cnt

    @pl.when(i == nsteps - 1)
    def _fini():
        loss_ref[...] = loss_ref[...] * (BETA_C / (nsteps * BL * DIM))
        avg = cnt_scr[...] * (1.0 / (nsteps * BL * N_STAGES))
        ent = jnp.sum(avg * jnp.log(avg + 1e-10), axis=1, keepdims=True)
        ppl_ref[...] = jnp.exp(-ent)


@jax.jit
def kernel(z, weight):
    b, c, h, w = z.shape
    nsteps = ROWS // BL
    z_flat = jnp.transpose(z, (0, 2, 3, 1)).reshape(ROWS, DIM)

    zq_flat, enc, idx, loss, ppl = pl.pallas_call(
        functools.partial(_vq_body, nsteps),
        grid=(nsteps,),
        in_specs=[
            pl.BlockSpec((BL, DIM), lambda i: (i, 0)),
            pl.BlockSpec((N_CODES, DIM), lambda i: (0, 0)),
        ],
        out_specs=[
            pl.BlockSpec((BL, DIM), lambda i: (i, 0)),
            pl.BlockSpec((N_STAGES, BL, N_CODES), lambda i: (0, i, 0)),
            pl.BlockSpec((BL, N_STAGES), lambda i: (i, 0)),
            pl.BlockSpec((1, 1), lambda i: (0, 0)),
            pl.BlockSpec((1, 1), lambda i: (0, 0)),
        ],
        out_shape=[
            jax.ShapeDtypeStruct((ROWS, DIM), jnp.float32),
            jax.ShapeDtypeStruct((N_STAGES, ROWS, N_CODES), jnp.float32),
            jax.ShapeDtypeStruct((ROWS, N_STAGES), jnp.int32),
            jax.ShapeDtypeStruct((1, 1), jnp.float32),
            jax.ShapeDtypeStruct((1, 1), jnp.float32),
        ],
        scratch_shapes=[pltpu.VMEM((1, N_CODES), jnp.float32)],
    )(z_flat, weight)

    z_q = jnp.transpose(zq_flat.reshape(b, h, w, DIM), (0, 3, 1, 2))
    encodings_cat = enc.reshape(N_STAGES * ROWS, N_CODES)
    indices_stack = jnp.transpose(idx.reshape(b, h, w, N_STAGES), (0, 3, 1, 2))
    return (z_q, loss[0, 0], ppl[0, 0], encodings_cat, indices_stack)
